# Initial kernel scaffold; baseline (speedup 1.0000x reference)
import jax
import jax.numpy as jnp
from jax.experimental import pallas as pl


def kernel(w):
    raise NotImplementedError("write your pallas kernel here")


# speedup vs baseline: 8.3527x; 8.3527x over previous
import jax
import jax.numpy as jnp
from jax.experimental import pallas as pl


_BC = 256


def _getmax_block(w_ref, o_ref):
    x = w_ref[:, :]
    a = jnp.abs(x)
    m = jnp.max(a, axis=0, keepdims=True)
    rows = jax.lax.broadcasted_iota(jnp.int32, x.shape, 0)
    masked_rows = jnp.where(a == m, rows, x.shape[0])
    first = jnp.min(masked_rows, axis=0, keepdims=True)
    o_ref[:, :] = jnp.where(rows == first, x, 0.0)


def kernel(w):
    n, mcols = w.shape
    grid = (mcols // _BC,)
    return pl.pallas_call(
        _getmax_block,
        grid=grid,
        in_specs=[pl.BlockSpec((n, _BC), lambda j: (0, j))],
        out_specs=pl.BlockSpec((n, _BC), lambda j: (0, j)),
        out_shape=jax.ShapeDtypeStruct((n, mcols), w.dtype),
    )(w)


# speedup vs baseline: 8.6119x; 1.0310x over previous
import jax
import jax.numpy as jnp
from jax.experimental import pallas as pl


_BC = 256


def _getmax_block(w_ref, o_ref):
    x = w_ref[:, :]
    a = jnp.abs(x)
    fi = jnp.argmax(a, axis=0)
    rows = jax.lax.broadcasted_iota(jnp.int32, x.shape, 0)
    o_ref[:, :] = jnp.where(rows == fi[None, :], x, 0.0)


def kernel(w):
    n, mcols = w.shape
    grid = (mcols // _BC,)
    return pl.pallas_call(
        _getmax_block,
        grid=grid,
        in_specs=[pl.BlockSpec((n, _BC), lambda j: (0, j))],
        out_specs=pl.BlockSpec((n, _BC), lambda j: (0, j)),
        out_shape=jax.ShapeDtypeStruct((n, mcols), w.dtype),
    )(w)
